# Initial kernel scaffold; baseline (speedup 1.0000x reference)
import jax
import jax.numpy as jnp
from jax.experimental import pallas as pl


def kernel(edge_index, user_emb, movie_emb, W_l1, b_l1, W_r1, W_l2, b_l2, W_r2):
    raise NotImplementedError("write your pallas kernel here")


# speedup vs baseline: 28.2169x; 28.2169x over previous
import functools

import jax
import jax.numpy as jnp
from jax import lax
from jax.experimental import pallas as pl
from jax.experimental.pallas import tpu as pltpu
from jax.experimental.pallas import tpu_sc as plsc

D = 16
L = 16
NC = 2
NS = 16
NW = NC * NS
CK = 128


def _seg_sum_rows(x, ei_r, np_rows, ch):
    pt = np_rows // NS
    zr = pt // 8
    mesh = plsc.VectorSubcoreMesh(core_axis_name="c", subcore_axis_name="s")

    @functools.partial(
        pl.kernel,
        out_type=(
            jax.ShapeDtypeStruct((NC, np_rows, D), jnp.float32),
            jax.ShapeDtypeStruct((NC, NS, np_rows // NS), jnp.float32),
        ),
        mesh=mesh,
        scratch_types=[
            pltpu.VMEM((56, CK), jnp.int32),
            pltpu.VMEM((56, CK), jnp.int32),
            pltpu.VMEM((CK, D), jnp.float32),
            pltpu.VMEM((CK, D), jnp.float32),
            pltpu.VMEM((CK,), jnp.float32),
            pltpu.VMEM((zr, D), jnp.float32),
            pltpu.VMEM((pt,), jnp.float32),
            pltpu.VMEM_SHARED((np_rows, D), jnp.float32),
            pltpu.VMEM_SHARED((np_rows,), jnp.float32),
            pltpu.SemaphoreType.DMA,
            pltpu.SemaphoreType.DMA,
        ],
        compiler_params=pltpu.CompilerParams(use_tc_tiling_on_sc=False),
    )
    def k(x_hbm, ei_hbm, psum_hbm, pcnt_hbm, src_v, dst_v, rows0, rows1,
          ones_v, zacc_v, zcnt_v, acc_sh, cnt_sh, sem0, sem1):
        c = lax.axis_index("c")
        s = lax.axis_index("s")
        wid = s * NC + c
        base = s * pt
        zv = jnp.zeros((L,), jnp.float32)

        def zb(i, _):
            zacc_v[i] = zv
            return 0
        lax.fori_loop(0, zr, zb, 0)

        def zc(i, _):
            zcnt_v[pl.ds(i * L, L)] = zv
            return 0
        lax.fori_loop(0, pt // L, zc, 0)

        ov = jnp.ones((L,), jnp.float32)
        for i in range(CK // L):
            ones_v[pl.ds(i * L, L)] = ov

        for j in range(8):
            pltpu.sync_copy(zacc_v, acc_sh.at[pl.ds(base + j * zr, zr)])
        pltpu.sync_copy(zcnt_v, cnt_sh.at[pl.ds(base, pt)])

        plsc.subcore_barrier()

        def slab(k2, _):
            pltpu.sync_copy(ei_hbm.at[0, wid, pl.ds(k2 * 56, 56)], src_v)
            pltpu.sync_copy(ei_hbm.at[1, wid, pl.ds(k2 * 56, 56)], dst_v)

            def body(jj, _):
                j0 = jj * 2
                j1 = j0 + 1
                d0 = pltpu.async_copy(x_hbm.at[src_v.at[j0]], rows0, sem0)
                d1 = pltpu.async_copy(x_hbm.at[src_v.at[j1]], rows1, sem1)
                d0.wait()
                pltpu.sync_copy(rows0, acc_sh.at[dst_v.at[j0]], add=True)
                pltpu.sync_copy(ones_v, cnt_sh.at[dst_v.at[j0]], add=True)
                d1.wait()
                pltpu.sync_copy(rows1, acc_sh.at[dst_v.at[j1]], add=True)
                pltpu.sync_copy(ones_v, cnt_sh.at[dst_v.at[j1]], add=True)
                return 0
            lax.fori_loop(0, 28, body, 0)
            return 0
        lax.fori_loop(0, ch // 56, slab, 0)
        plsc.subcore_barrier()

        pltpu.sync_copy(acc_sh.at[pl.ds(base, pt)],
                        psum_hbm.at[c, pl.ds(base, pt)])
        pltpu.sync_copy(cnt_sh.at[pl.ds(base, pt)], pcnt_hbm.at[c, s])

    return k(x, ei_r)


def _seg_sum_scalars(t_pad, ei_r, np_rows, ch):
    slr = 56
    nsl = ch // slr
    mesh = plsc.VectorSubcoreMesh(core_axis_name="c", subcore_axis_name="s")

    @functools.partial(
        pl.kernel,
        out_type=jax.ShapeDtypeStruct((NW, 1, np_rows), jnp.float32),
        mesh=mesh,
        scratch_types=[
            pltpu.VMEM((slr, CK), jnp.int32),
            pltpu.VMEM((slr, CK), jnp.int32),
            pltpu.VMEM((np_rows,), jnp.float32),
            pltpu.VMEM((np_rows,), jnp.float32),
        ],
        compiler_params=pltpu.CompilerParams(needs_layout_passes=False),
    )
    def k(t_hbm, ei_hbm, out_hbm, src_sl, dst_sl, t_v, acc_v):
        c = lax.axis_index("c")
        s = lax.axis_index("s")
        wid = s * NC + c
        zv = jnp.zeros((L,), jnp.float32)

        def za(i, _):
            acc_v[pl.ds(i * L, L)] = zv
            return 0
        lax.fori_loop(0, np_rows // L, za, 0)

        pltpu.sync_copy(t_hbm, t_v)

        def slab(k2, _):
            pltpu.sync_copy(ei_hbm.at[0, wid, pl.ds(k2 * slr, slr)], src_sl)
            pltpu.sync_copy(ei_hbm.at[1, wid, pl.ds(k2 * slr, slr)], dst_sl)

            def row(r, _):
                for v in range(CK // L):
                    sv = src_sl[r, pl.ds(v * L, L)]
                    dv = dst_sl[r, pl.ds(v * L, L)]
                    vals = plsc.load_gather(t_v, [sv])
                    plsc.addupdate_scatter(acc_v, [dv], vals)
                return 0
            lax.fori_loop(0, slr, row, 0)
            return 0
        lax.fori_loop(0, nsl, slab, 0)

        pltpu.sync_copy(acc_v, out_hbm.at[wid, 0])

    return k(t_pad, ei_r)


def _dense_mid(p, cnt3, x, wl1, bl1, wr1, wl2, bl2, wr2):
    n = x.shape[0]
    b = 2000
    g = n // b

    def body(p_ref, c_ref, x_ref, wl1_ref, bl1_ref, wr1_ref, wl2_ref,
             bl2_ref, wr2_ref, t_ref, r2_ref, den_ref):
        agg = p_ref[0] + p_ref[1]
        den = jnp.maximum(c_ref[0] + c_ref[1], 1.0)
        xb = x_ref[...]
        y = (jnp.dot(agg / den, wl1_ref[...],
                     preferred_element_type=jnp.float32)
             + bl1_ref[...]
             + jnp.dot(xb, wr1_ref[...], preferred_element_type=jnp.float32))
        y = jnp.maximum(y, 0.0)
        t_ref[...] = jnp.dot(y, wl2_ref[...],
                             preferred_element_type=jnp.float32)
        r2_ref[...] = (jnp.dot(y, wr2_ref[...],
                               preferred_element_type=jnp.float32)
                       + bl2_ref[...])
        den_ref[...] = den

    full = lambda i: (0, 0)
    return pl.pallas_call(
        body,
        grid=(g,),
        in_specs=[
            pl.BlockSpec((NC, b, D), lambda i: (0, i, 0)),
            pl.BlockSpec((NC, b, 1), lambda i: (0, i, 0)),
            pl.BlockSpec((b, D), lambda i: (i, 0)),
            pl.BlockSpec((D, D), full),
            pl.BlockSpec((1, D), full),
            pl.BlockSpec((D, D), full),
            pl.BlockSpec((D, 1), full),
            pl.BlockSpec((1, 1), full),
            pl.BlockSpec((D, 1), full),
        ],
        out_specs=[
            pl.BlockSpec((b, 1), lambda i: (i, 0)),
            pl.BlockSpec((b, 1), lambda i: (i, 0)),
            pl.BlockSpec((b, 1), lambda i: (i, 0)),
        ],
        out_shape=[
            jax.ShapeDtypeStruct((n, 1), jnp.float32),
            jax.ShapeDtypeStruct((n, 1), jnp.float32),
            jax.ShapeDtypeStruct((n, 1), jnp.float32),
        ],
    )(p, cnt3, x, wl1, bl1, wr1, wl2, bl2, wr2)


def _dense_out(q2, den2, r22):
    n = den2.shape[1]

    def body(q_ref, den_ref, r2_ref, o_ref):
        q = jnp.sum(q_ref[...], axis=0, keepdims=True)
        o_ref[...] = q / den_ref[...] + r2_ref[...]

    return pl.pallas_call(
        body,
        out_shape=jax.ShapeDtypeStruct((1, n), jnp.float32),
    )(q2, den2, r22)


def kernel(edge_index, user_emb, movie_emb, W_l1, b_l1, W_r1, W_l2, b_l2,
           W_r2):
    x = jnp.concatenate([user_emb, movie_emb], axis=0)
    n = x.shape[0]
    e = edge_index.shape[1]

    np_rows = (n + 1 + NS * L - 1) // (NS * L) * (NS * L)
    ch = (e + NW * CK - 1) // (NW * CK)
    ch = -(-ch // 56) * 56
    e_pad = NW * ch * CK

    ei = edge_index.astype(jnp.int32)
    pad = e_pad - e
    ei = jnp.concatenate(
        [ei, jnp.stack([jnp.zeros((pad,), jnp.int32),
                        jnp.full((pad,), n, jnp.int32)])], axis=1)
    ei_r = ei.reshape(2, NW, ch, CK)

    psum, pcnt = _seg_sum_rows(x, ei_r, np_rows, ch)
    pcnt = pcnt.reshape(NC, np_rows)
    t, r2, den = _dense_mid(psum[:, :n, :], pcnt[:, :n, None], x,
                            W_l1, b_l1.reshape(1, D), W_r1,
                            W_l2, b_l2.reshape(1, 1), W_r2)
    t_pad = jnp.pad(t[:, 0], (0, np_rows - n))
    q = _seg_sum_scalars(t_pad, ei_r, np_rows, ch).reshape(NW, np_rows)
    out = _dense_out(q[:, :n], den.reshape(1, n), r2.reshape(1, n))
    return out.reshape(n, 1)


# speedup vs baseline: 29.2129x; 1.0353x over previous
import functools

import jax
import jax.numpy as jnp
from jax import lax
from jax.experimental import pallas as pl
from jax.experimental.pallas import tpu as pltpu
from jax.experimental.pallas import tpu_sc as plsc

D = 16
L = 16
NC = 2
NS = 16
NW = NC * NS
CK = 128


def _chunk_split(nch):
    g8 = nch // 8
    gb = g8 // NW
    cb = gb * 8
    grem = g8 % NW
    tail0 = NW * cb + grem * 8
    tail = nch - tail0
    sl = 8
    for cand in range(56, 7, -8):
        if cb % cand == 0:
            sl = cand
            break
    return cb, sl, grem, tail0, tail


def _seg_sum_rows(x, ei_r, np_rows):
    n = x.shape[0]
    nch = ei_r.shape[1]
    cb, sl, grem, tail0, tail = _chunk_split(nch)
    pt = np_rows // NS
    lt = n - (NS - 1) * pt
    zr = pt // 8
    mesh = plsc.VectorSubcoreMesh(core_axis_name="c", subcore_axis_name="s")

    @functools.partial(
        pl.kernel,
        out_type=(
            jax.ShapeDtypeStruct((NC, n, D), jnp.float32),
            jax.ShapeDtypeStruct((NC, 1, n), jnp.float32),
        ),
        mesh=mesh,
        scratch_types=[
            pltpu.VMEM((sl, CK), jnp.int32),
            pltpu.VMEM((sl, CK), jnp.int32),
            pltpu.VMEM((CK, D), jnp.float32),
            pltpu.VMEM((CK, D), jnp.float32),
            pltpu.VMEM((CK,), jnp.float32),
            pltpu.VMEM((zr, D), jnp.float32),
            pltpu.VMEM((pt,), jnp.float32),
            pltpu.VMEM_SHARED((np_rows, D), jnp.float32),
            pltpu.VMEM_SHARED((np_rows,), jnp.float32),
            pltpu.SemaphoreType.DMA,
            pltpu.SemaphoreType.DMA,
        ],
        compiler_params=pltpu.CompilerParams(use_tc_tiling_on_sc=False),
    )
    def k(x_hbm, ei_hbm, psum_hbm, pcnt_hbm, src_v, dst_v, rows0, rows1,
          ones_v, zacc_v, zcnt_v, acc_sh, cnt_sh, sem0, sem1):
        c = lax.axis_index("c")
        s = lax.axis_index("s")
        wid = s * NC + c
        base = s * pt
        zv = jnp.zeros((L,), jnp.float32)

        def zb(i, _):
            zacc_v[i] = zv
            return 0
        lax.fori_loop(0, zr, zb, 0)

        def zc(i, _):
            zcnt_v[pl.ds(i * L, L)] = zv
            return 0
        lax.fori_loop(0, pt // L, zc, 0)

        ov = jnp.ones((L,), jnp.float32)
        for i in range(CK // L):
            ones_v[pl.ds(i * L, L)] = ov

        for j in range(8):
            pltpu.sync_copy(zacc_v, acc_sh.at[pl.ds(base + j * zr, zr)])
        pltpu.sync_copy(zcnt_v, cnt_sh.at[pl.ds(base, pt)])
        plsc.subcore_barrier()

        lo = wid * cb

        def pair_body(jj, _):
            j0 = jj * 2
            j1 = j0 + 1
            d0 = pltpu.async_copy(x_hbm.at[src_v.at[j0]], rows0, sem0)
            d1 = pltpu.async_copy(x_hbm.at[src_v.at[j1]], rows1, sem1)
            d0.wait()
            pltpu.sync_copy(rows0, acc_sh.at[dst_v.at[j0]], add=True)
            pltpu.sync_copy(ones_v, cnt_sh.at[dst_v.at[j0]], add=True)
            d1.wait()
            pltpu.sync_copy(rows1, acc_sh.at[dst_v.at[j1]], add=True)
            pltpu.sync_copy(ones_v, cnt_sh.at[dst_v.at[j1]], add=True)
            return 0

        def slab(k2, _):
            off = lo + k2 * sl
            pltpu.sync_copy(ei_hbm.at[0, pl.ds(off, sl)], src_v)
            pltpu.sync_copy(ei_hbm.at[1, pl.ds(off, sl)], dst_v)
            lax.fori_loop(0, sl // 2, pair_body, 0)
            return 0
        lax.fori_loop(0, cb // sl, slab, 0)

        if grem:
            @pl.when(wid < grem)
            def _extra():
                xo = NW * cb + wid * 8
                pltpu.sync_copy(ei_hbm.at[0, pl.ds(xo, 8)],
                                src_v.at[pl.ds(0, 8)])
                pltpu.sync_copy(ei_hbm.at[1, pl.ds(xo, 8)],
                                dst_v.at[pl.ds(0, 8)])
                lax.fori_loop(0, 4, pair_body, 0)

        if tail:
            @pl.when(wid == grem)
            def _tail():
                pltpu.sync_copy(ei_hbm.at[0, pl.ds(tail0, tail)],
                                src_v.at[pl.ds(0, tail)])
                pltpu.sync_copy(ei_hbm.at[1, pl.ds(tail0, tail)],
                                dst_v.at[pl.ds(0, tail)])
                for j in range(tail):
                    pltpu.async_copy(
                        x_hbm.at[src_v.at[j]], rows0, sem0).wait()
                    pltpu.sync_copy(rows0, acc_sh.at[dst_v.at[j]], add=True)
                    pltpu.sync_copy(ones_v, cnt_sh.at[dst_v.at[j]], add=True)
        plsc.subcore_barrier()

        @pl.when(s < NS - 1)
        def _full():
            pltpu.sync_copy(acc_sh.at[pl.ds(base, pt)],
                            psum_hbm.at[c, pl.ds(base, pt)])
            pltpu.sync_copy(cnt_sh.at[pl.ds(base, pt)],
                            pcnt_hbm.at[c, 0, pl.ds(base, pt)])

        @pl.when(s == NS - 1)
        def _last():
            pltpu.sync_copy(acc_sh.at[pl.ds(base, lt)],
                            psum_hbm.at[c, pl.ds(base, lt)])
            pltpu.sync_copy(cnt_sh.at[pl.ds(base, lt)],
                            pcnt_hbm.at[c, 0, pl.ds(base, lt)])

    return k(x, ei_r)


def _seg_sum_scalars(t, ei_r):
    n = t.shape[0]
    a2 = (n + 1 + L - 1) // L * L
    nch = ei_r.shape[1]
    cb, sl, grem, tail0, tail = _chunk_split(nch)
    mesh = plsc.VectorSubcoreMesh(core_axis_name="c", subcore_axis_name="s")

    @functools.partial(
        pl.kernel,
        out_type=jax.ShapeDtypeStruct((NW, 1, n), jnp.float32),
        mesh=mesh,
        scratch_types=[
            pltpu.VMEM((sl, CK), jnp.int32),
            pltpu.VMEM((sl, CK), jnp.int32),
            pltpu.VMEM((n,), jnp.float32),
            pltpu.VMEM((a2,), jnp.float32),
        ],
        compiler_params=pltpu.CompilerParams(needs_layout_passes=False),
    )
    def k(t_hbm, ei_hbm, out_hbm, src_sl, dst_sl, t_v, acc_v):
        c = lax.axis_index("c")
        s = lax.axis_index("s")
        wid = s * NC + c
        zv = jnp.zeros((L,), jnp.float32)

        def za(i, _):
            acc_v[pl.ds(i * L, L)] = zv
            return 0
        lax.fori_loop(0, a2 // L, za, 0)

        pltpu.sync_copy(t_hbm, t_v)
        lo = wid * cb

        def row(r, _):
            for v in range(CK // L):
                sv = src_sl[r, pl.ds(v * L, L)]
                dv = dst_sl[r, pl.ds(v * L, L)]
                vals = plsc.load_gather(t_v, [sv])
                plsc.addupdate_scatter(acc_v, [dv], vals)
            return 0

        def slab(k2, _):
            off = lo + k2 * sl
            pltpu.sync_copy(ei_hbm.at[0, pl.ds(off, sl)], src_sl)
            pltpu.sync_copy(ei_hbm.at[1, pl.ds(off, sl)], dst_sl)
            lax.fori_loop(0, sl, row, 0)
            return 0
        lax.fori_loop(0, cb // sl, slab, 0)

        if grem:
            @pl.when(wid < grem)
            def _extra():
                xo = NW * cb + wid * 8
                pltpu.sync_copy(ei_hbm.at[0, pl.ds(xo, 8)],
                                src_sl.at[pl.ds(0, 8)])
                pltpu.sync_copy(ei_hbm.at[1, pl.ds(xo, 8)],
                                dst_sl.at[pl.ds(0, 8)])
                lax.fori_loop(0, 8, row, 0)

        if tail:
            @pl.when(wid == grem)
            def _tail():
                pltpu.sync_copy(ei_hbm.at[0, pl.ds(tail0, tail)],
                                src_sl.at[pl.ds(0, tail)])
                pltpu.sync_copy(ei_hbm.at[1, pl.ds(tail0, tail)],
                                dst_sl.at[pl.ds(0, tail)])
                lax.fori_loop(0, tail, row, 0)

        pltpu.sync_copy(acc_v.at[pl.ds(0, n)], out_hbm.at[wid, 0])

    return k(t, ei_r)


def _dense_mid(p, cnt3, x, wl1, bl1, wr1, wl2, bl2, wr2):
    n = x.shape[0]
    b = 2000
    g = n // b

    def body(p_ref, c_ref, x_ref, wl1_ref, bl1_ref, wr1_ref, wl2_ref,
             bl2_ref, wr2_ref, t_ref, r2_ref, den_ref):
        agg = p_ref[0] + p_ref[1]
        den = jnp.maximum(c_ref[0] + c_ref[1], 1.0)
        xb = x_ref[...]
        y = (jnp.dot(agg / den, wl1_ref[...],
                     preferred_element_type=jnp.float32)
             + bl1_ref[...]
             + jnp.dot(xb, wr1_ref[...], preferred_element_type=jnp.float32))
        y = jnp.maximum(y, 0.0)
        t_ref[...] = jnp.dot(y, wl2_ref[...],
                             preferred_element_type=jnp.float32)
        r2_ref[...] = (jnp.dot(y, wr2_ref[...],
                               preferred_element_type=jnp.float32)
                       + bl2_ref[...])
        den_ref[...] = den

    full = lambda i: (0, 0)
    return pl.pallas_call(
        body,
        grid=(g,),
        in_specs=[
            pl.BlockSpec((NC, b, D), lambda i: (0, i, 0)),
            pl.BlockSpec((NC, b, 1), lambda i: (0, i, 0)),
            pl.BlockSpec((b, D), lambda i: (i, 0)),
            pl.BlockSpec((D, D), full),
            pl.BlockSpec((1, D), full),
            pl.BlockSpec((D, D), full),
            pl.BlockSpec((D, 1), full),
            pl.BlockSpec((1, 1), full),
            pl.BlockSpec((D, 1), full),
        ],
        out_specs=[
            pl.BlockSpec((b, 1), lambda i: (i, 0)),
            pl.BlockSpec((b, 1), lambda i: (i, 0)),
            pl.BlockSpec((b, 1), lambda i: (i, 0)),
        ],
        out_shape=[
            jax.ShapeDtypeStruct((n, 1), jnp.float32),
            jax.ShapeDtypeStruct((n, 1), jnp.float32),
            jax.ShapeDtypeStruct((n, 1), jnp.float32),
        ],
    )(p, cnt3, x, wl1, bl1, wr1, wl2, bl2, wr2)


def _dense_out(q2, den2, r22):
    n = den2.shape[1]

    def body(q_ref, den_ref, r2_ref, o_ref):
        q = jnp.sum(q_ref[...], axis=0, keepdims=True)
        o_ref[...] = q / den_ref[...] + r2_ref[...]

    return pl.pallas_call(
        body,
        out_shape=jax.ShapeDtypeStruct((1, n), jnp.float32),
    )(q2, den2, r22)


def kernel(edge_index, user_emb, movie_emb, W_l1, b_l1, W_r1, W_l2, b_l2,
           W_r2):
    x = jnp.concatenate([user_emb, movie_emb], axis=0)
    n = x.shape[0]
    e = edge_index.shape[1]

    ei = edge_index.astype(jnp.int32)
    if e % CK:
        pad = CK - e % CK
        ei = jnp.concatenate(
            [ei, jnp.stack([jnp.zeros((pad,), jnp.int32),
                            jnp.full((pad,), n, jnp.int32)])], axis=1)
    ei_r = ei.reshape(2, -1, CK)

    np_rows = (n + 1 + NS * L - 1) // (NS * L) * (NS * L)

    psum, pcnt = _seg_sum_rows(x, ei_r, np_rows)
    t, r2, den = _dense_mid(psum, pcnt.reshape(NC, n, 1), x,
                            W_l1, b_l1.reshape(1, D), W_r1,
                            W_l2, b_l2.reshape(1, 1), W_r2)
    q = _seg_sum_scalars(t.reshape(n), ei_r).reshape(NW, n)
    out = _dense_out(q, den.reshape(1, n), r2.reshape(1, n))
    return out.reshape(n, 1)


# speedup vs baseline: 38.4011x; 1.3145x over previous
import functools

import jax
import jax.numpy as jnp
from jax import lax
from jax.experimental import pallas as pl
from jax.experimental.pallas import tpu as pltpu
from jax.experimental.pallas import tpu_sc as plsc

D = 16
L = 16
NC = 2
NS = 16
NW = NC * NS
CK = 128


def _chunk_split(nch):
    g8 = nch // 8
    gb = g8 // NW
    cb = gb * 8
    grem = g8 % NW
    tail0 = NW * cb + grem * 8
    tail = nch - tail0
    sl = 8
    for cand in range(56, 7, -8):
        if cb % cand == 0:
            sl = cand
            break
    return cb, sl, grem, tail0, tail


def _seg_sum_rows(x, ei_r, np_rows):
    n = x.shape[0]
    nch = ei_r.shape[1]
    cb, sl, grem, tail0, tail = _chunk_split(nch)
    pt = np_rows // NS
    lt = n - (NS - 1) * pt
    zr = pt // 8
    mesh = plsc.VectorSubcoreMesh(core_axis_name="c", subcore_axis_name="s")

    @functools.partial(
        pl.kernel,
        out_type=(
            jax.ShapeDtypeStruct((NC, n, D), jnp.float32),
            jax.ShapeDtypeStruct((NC, 1, n), jnp.float32),
        ),
        mesh=mesh,
        scratch_types=[
            pltpu.VMEM((sl, CK), jnp.int32),
            pltpu.VMEM((sl, CK), jnp.int32),
            pltpu.VMEM((CK, D), jnp.float32),
            pltpu.VMEM((CK, D), jnp.float32),
            pltpu.VMEM((CK,), jnp.float32),
            pltpu.VMEM((zr, D), jnp.float32),
            pltpu.VMEM((pt,), jnp.float32),
            pltpu.VMEM_SHARED((np_rows, D), jnp.float32),
            pltpu.VMEM_SHARED((np_rows,), jnp.float32),
            pltpu.SemaphoreType.DMA,
            pltpu.SemaphoreType.DMA,
        ],
        compiler_params=pltpu.CompilerParams(use_tc_tiling_on_sc=False),
    )
    def k(x_hbm, ei_hbm, psum_hbm, pcnt_hbm, src_v, dst_v, rows0, rows1,
          ones_v, zacc_v, zcnt_v, acc_sh, cnt_sh, sem0, sem1):
        c = lax.axis_index("c")
        s = lax.axis_index("s")
        wid = s * NC + c
        base = s * pt
        zv = jnp.zeros((L,), jnp.float32)

        def zb(i, _):
            zacc_v[i] = zv
            return 0
        lax.fori_loop(0, zr, zb, 0)

        def zc(i, _):
            zcnt_v[pl.ds(i * L, L)] = zv
            return 0
        lax.fori_loop(0, pt // L, zc, 0)

        ov = jnp.ones((L,), jnp.float32)
        for i in range(CK // L):
            ones_v[pl.ds(i * L, L)] = ov

        for j in range(8):
            pltpu.sync_copy(zacc_v, acc_sh.at[pl.ds(base + j * zr, zr)])
        pltpu.sync_copy(zcnt_v, cnt_sh.at[pl.ds(base, pt)])
        plsc.subcore_barrier()

        lo = wid * cb

        def pair_body(jj, _):
            j0 = jj * 2
            j1 = j0 + 1
            d0 = pltpu.async_copy(x_hbm.at[src_v.at[j0]], rows0, sem0)
            d1 = pltpu.async_copy(x_hbm.at[src_v.at[j1]], rows1, sem1)
            d0.wait()
            pltpu.sync_copy(rows0, acc_sh.at[dst_v.at[j0]], add=True)
            pltpu.sync_copy(ones_v, cnt_sh.at[dst_v.at[j0]], add=True)
            d1.wait()
            pltpu.sync_copy(rows1, acc_sh.at[dst_v.at[j1]], add=True)
            pltpu.sync_copy(ones_v, cnt_sh.at[dst_v.at[j1]], add=True)
            return 0

        def slab(k2, _):
            off = lo + k2 * sl
            pltpu.sync_copy(ei_hbm.at[0, pl.ds(off, sl)], src_v)
            pltpu.sync_copy(ei_hbm.at[1, pl.ds(off, sl)], dst_v)
            lax.fori_loop(0, sl // 2, pair_body, 0)
            return 0
        lax.fori_loop(0, cb // sl, slab, 0)

        if grem:
            @pl.when(wid < grem)
            def _extra():
                xo = NW * cb + wid * 8
                pltpu.sync_copy(ei_hbm.at[0, pl.ds(xo, 8)],
                                src_v.at[pl.ds(0, 8)])
                pltpu.sync_copy(ei_hbm.at[1, pl.ds(xo, 8)],
                                dst_v.at[pl.ds(0, 8)])
                lax.fori_loop(0, 4, pair_body, 0)

        if tail:
            @pl.when(wid == grem)
            def _tail():
                pltpu.sync_copy(ei_hbm.at[0, pl.ds(tail0, tail)],
                                src_v.at[pl.ds(0, tail)])
                pltpu.sync_copy(ei_hbm.at[1, pl.ds(tail0, tail)],
                                dst_v.at[pl.ds(0, tail)])
                for j in range(tail):
                    pltpu.async_copy(
                        x_hbm.at[src_v.at[j]], rows0, sem0).wait()
                    pltpu.sync_copy(rows0, acc_sh.at[dst_v.at[j]], add=True)
                    pltpu.sync_copy(ones_v, cnt_sh.at[dst_v.at[j]], add=True)
        plsc.subcore_barrier()

        @pl.when(s < NS - 1)
        def _full():
            pltpu.sync_copy(acc_sh.at[pl.ds(base, pt)],
                            psum_hbm.at[c, pl.ds(base, pt)])
            pltpu.sync_copy(cnt_sh.at[pl.ds(base, pt)],
                            pcnt_hbm.at[c, 0, pl.ds(base, pt)])

        @pl.when(s == NS - 1)
        def _last():
            pltpu.sync_copy(acc_sh.at[pl.ds(base, lt)],
                            psum_hbm.at[c, pl.ds(base, lt)])
            pltpu.sync_copy(cnt_sh.at[pl.ds(base, lt)],
                            pcnt_hbm.at[c, 0, pl.ds(base, lt)])

    return k(x, ei_r)


def _seg_sum_scalars(t, ei_r):
    n = t.shape[0]
    a2 = (n + 1 + L - 1) // L * L
    nch = ei_r.shape[1]
    cb, sl, grem, tail0, tail = _chunk_split(nch)
    mesh = plsc.VectorSubcoreMesh(core_axis_name="c", subcore_axis_name="s")

    @functools.partial(
        pl.kernel,
        out_type=jax.ShapeDtypeStruct((NW, 1, n), jnp.float32),
        mesh=mesh,
        scratch_types=[
            pltpu.VMEM((sl, CK), jnp.int32),
            pltpu.VMEM((sl, CK), jnp.int32),
            pltpu.VMEM((n,), jnp.float32),
            pltpu.VMEM((a2,), jnp.float32),
        ],
        compiler_params=pltpu.CompilerParams(needs_layout_passes=False),
    )
    def k(t_hbm, ei_hbm, out_hbm, src_sl, dst_sl, t_v, acc_v):
        c = lax.axis_index("c")
        s = lax.axis_index("s")
        wid = s * NC + c
        zv = jnp.zeros((L,), jnp.float32)

        def za(i, _):
            acc_v[pl.ds(i * L, L)] = zv
            return 0
        lax.fori_loop(0, a2 // L, za, 0)

        pltpu.sync_copy(t_hbm, t_v)
        lo = wid * cb

        def row(r, _):
            for v in range(CK // L):
                sv = src_sl[r, pl.ds(v * L, L)]
                dv = dst_sl[r, pl.ds(v * L, L)]
                vals = plsc.load_gather(t_v, [sv])
                plsc.addupdate_scatter(acc_v, [dv], vals)
            return 0

        def slab(k2, _):
            off = lo + k2 * sl
            pltpu.sync_copy(ei_hbm.at[0, pl.ds(off, sl)], src_sl)
            pltpu.sync_copy(ei_hbm.at[1, pl.ds(off, sl)], dst_sl)
            lax.fori_loop(0, sl, row, 0)
            return 0
        lax.fori_loop(0, cb // sl, slab, 0)

        if grem:
            @pl.when(wid < grem)
            def _extra():
                xo = NW * cb + wid * 8
                pltpu.sync_copy(ei_hbm.at[0, pl.ds(xo, 8)],
                                src_sl.at[pl.ds(0, 8)])
                pltpu.sync_copy(ei_hbm.at[1, pl.ds(xo, 8)],
                                dst_sl.at[pl.ds(0, 8)])
                lax.fori_loop(0, 8, row, 0)

        if tail:
            @pl.when(wid == grem)
            def _tail():
                pltpu.sync_copy(ei_hbm.at[0, pl.ds(tail0, tail)],
                                src_sl.at[pl.ds(0, tail)])
                pltpu.sync_copy(ei_hbm.at[1, pl.ds(tail0, tail)],
                                dst_sl.at[pl.ds(0, tail)])
                lax.fori_loop(0, tail, row, 0)

        pltpu.sync_copy(acc_v.at[pl.ds(0, n)], out_hbm.at[wid, 0])

    return k(t, ei_r)


def _dense_mid(psum_p, den_p, x_p, wl1k, bl1k, wr1k, wl2k, wr2k, bl2k):
    n8 = x_p.shape[0]

    def body(p_ref, den_ref, x_ref, wl1_ref, bl1_ref, wr1_ref, wl2_ref,
             wr2_ref, bl2_ref, t_ref, r2_ref):
        agg = (p_ref[0] + p_ref[1]) / den_ref[...]
        y = (jnp.dot(agg, wl1_ref[...], preferred_element_type=jnp.float32)
             + bl1_ref[...]
             + jnp.dot(x_ref[...], wr1_ref[...],
                       preferred_element_type=jnp.float32))
        y = jnp.maximum(y, 0.0)
        t_ref[...] = jnp.dot(y, wl2_ref[...],
                             preferred_element_type=jnp.float32)
        r2_ref[...] = (jnp.dot(y, wr2_ref[...],
                               preferred_element_type=jnp.float32)
                       + bl2_ref[...])

    return pl.pallas_call(
        body,
        out_shape=[
            jax.ShapeDtypeStruct((n8, 8), jnp.float32),
            jax.ShapeDtypeStruct((n8, 8), jnp.float32),
        ],
    )(psum_p, den_p, x_p, wl1k, bl1k, wr1k, wl2k, wr2k, bl2k)


def _dense_out(q2, pcnt2, r2row):
    n = q2.shape[1]

    def body(q_ref, c_ref, r2_ref, o_ref):
        q = jnp.sum(q_ref[...], axis=0, keepdims=True)
        den = jnp.maximum(c_ref[0] + c_ref[1], 1.0)
        o_ref[...] = q / den + r2_ref[...]

    return pl.pallas_call(
        body,
        out_shape=jax.ShapeDtypeStruct((1, n), jnp.float32),
    )(q2, pcnt2, r2row)


def kernel(edge_index, user_emb, movie_emb, W_l1, b_l1, W_r1, W_l2, b_l2,
           W_r2):
    x = jnp.concatenate([user_emb, movie_emb], axis=0)
    n = x.shape[0]
    e = edge_index.shape[1]

    ei = edge_index.astype(jnp.int32)
    if e % CK:
        pad = CK - e % CK
        ei = jnp.concatenate(
            [ei, jnp.stack([jnp.zeros((pad,), jnp.int32),
                            jnp.full((pad,), n, jnp.int32)])], axis=1)
    ei_r = ei.reshape(2, -1, CK)

    np_rows = (n + 1 + NS * L - 1) // (NS * L) * (NS * L)

    psum, pcnt = _seg_sum_rows(x, ei_r, np_rows)

    eye = jnp.eye(8, dtype=jnp.float32)
    denrow = jnp.maximum(pcnt[0, 0] + pcnt[1, 0], 1.0)
    den_p = jnp.broadcast_to(denrow[:, None], (n, D)).reshape(n // 8, 8 * D)
    t8, r28 = _dense_mid(
        psum.reshape(NC, n // 8, 8 * D), den_p, x.reshape(n // 8, 8 * D),
        jnp.kron(eye, W_l1), jnp.tile(b_l1, 8).reshape(1, 8 * D),
        jnp.kron(eye, W_r1), jnp.kron(eye, W_l2), jnp.kron(eye, W_r2),
        jnp.broadcast_to(b_l2.reshape(1, 1), (1, 8)))

    q = _seg_sum_scalars(t8.reshape(n), ei_r).reshape(NW, n)
    out = _dense_out(q, pcnt, r28.reshape(1, n))
    return out.reshape(n, 1)


# speedup vs baseline: 42.3643x; 1.1032x over previous
import functools

import jax
import jax.numpy as jnp
from jax import lax
from jax.experimental import pallas as pl
from jax.experimental.pallas import tpu as pltpu
from jax.experimental.pallas import tpu_sc as plsc

D = 16
L = 16
NC = 2
NS = 16
NW = NC * NS
CK = 128


def _chunk_split(nch):
    g8 = nch // 8
    gb = g8 // NW
    cb = gb * 8
    grem = g8 % NW
    tail0 = NW * cb + grem * 8
    tail = nch - tail0
    sl = 8
    for cand in range(56, 7, -8):
        if cb % cand == 0:
            sl = cand
            break
    return cb, sl, grem, tail0, tail


def _seg_sum_rows(x, ei_r, np_rows):
    n = x.shape[0]
    a2 = (n + 1 + L - 1) // L * L
    nch = ei_r.shape[1]
    cb, sl, grem, tail0, tail = _chunk_split(nch)
    pt = np_rows // NS
    lt = n - (NS - 1) * pt
    zr = pt // 8
    mesh = plsc.VectorSubcoreMesh(core_axis_name="c", subcore_axis_name="s")

    @functools.partial(
        pl.kernel,
        out_type=(
            jax.ShapeDtypeStruct((NC, n, D), jnp.float32),
            jax.ShapeDtypeStruct((NW, 1, n), jnp.float32),
        ),
        mesh=mesh,
        scratch_types=[
            pltpu.VMEM((sl, CK), jnp.int32),
            pltpu.VMEM((sl, CK), jnp.int32),
            pltpu.VMEM((CK, D), jnp.float32),
            pltpu.VMEM((CK, D), jnp.float32),
            pltpu.VMEM((CK, D), jnp.float32),
            pltpu.VMEM((CK, D), jnp.float32),
            pltpu.VMEM((zr, D), jnp.float32),
            pltpu.VMEM((a2,), jnp.float32),
            pltpu.VMEM_SHARED((np_rows, D), jnp.float32),
            pltpu.SemaphoreType.DMA,
            pltpu.SemaphoreType.DMA,
            pltpu.SemaphoreType.DMA,
            pltpu.SemaphoreType.DMA,
        ],
        compiler_params=pltpu.CompilerParams(use_tc_tiling_on_sc=False,
                                             needs_layout_passes=False),
    )
    def k(x_hbm, ei_hbm, psum_hbm, pcnt_hbm, src_v, dst_v, rows0, rows1,
          rows2, rows3, zacc_v, cnt_v, acc_sh, sem0, sem1, sem2, sem3):
        c = lax.axis_index("c")
        s = lax.axis_index("s")
        wid = s * NC + c
        base = s * pt
        zv = jnp.zeros((L,), jnp.float32)
        ov = jnp.ones((L,), jnp.float32)
        rows = (rows0, rows1, rows2, rows3)
        sems = (sem0, sem1, sem2, sem3)

        def zb(i, _):
            zacc_v[i] = zv
            return 0
        lax.fori_loop(0, zr, zb, 0)

        def zc(i, _):
            cnt_v[pl.ds(i * L, L)] = zv
            return 0
        lax.fori_loop(0, a2 // L, zc, 0)

        for j in range(8):
            pltpu.sync_copy(zacc_v, acc_sh.at[pl.ds(base + j * zr, zr)])
        plsc.subcore_barrier()

        lo = wid * cb

        def one_chunk(j, rbuf, sem):
            pltpu.async_copy(x_hbm.at[src_v.at[j]], rbuf, sem).wait()
            pltpu.sync_copy(rbuf, acc_sh.at[dst_v.at[j]], add=True)
            for v in range(CK // L):
                dv = dst_v[j, pl.ds(v * L, L)]
                plsc.addupdate_scatter(cnt_v, [dv], ov)

        def quad(qq, _):
            j = qq * 4
            ds = [pltpu.async_copy(x_hbm.at[src_v.at[j + b]], rows[b],
                                   sems[b]) for b in range(4)]
            for b in range(4):
                ds[b].wait()
                pltpu.sync_copy(rows[b], acc_sh.at[dst_v.at[j + b]],
                                add=True)
                for v in range(CK // L):
                    dv = dst_v[j + b, pl.ds(v * L, L)]
                    plsc.addupdate_scatter(cnt_v, [dv], ov)
            return 0

        def slab(k2, _):
            off = lo + k2 * sl
            pltpu.sync_copy(ei_hbm.at[0, pl.ds(off, sl)], src_v)
            pltpu.sync_copy(ei_hbm.at[1, pl.ds(off, sl)], dst_v)
            lax.fori_loop(0, sl // 4, quad, 0)
            return 0
        lax.fori_loop(0, cb // sl, slab, 0)

        if grem:
            @pl.when(wid < grem)
            def _extra():
                xo = NW * cb + wid * 8
                pltpu.sync_copy(ei_hbm.at[0, pl.ds(xo, 8)],
                                src_v.at[pl.ds(0, 8)])
                pltpu.sync_copy(ei_hbm.at[1, pl.ds(xo, 8)],
                                dst_v.at[pl.ds(0, 8)])
                lax.fori_loop(0, 2, quad, 0)

        if tail:
            @pl.when(wid == grem)
            def _tail():
                pltpu.sync_copy(ei_hbm.at[0, pl.ds(tail0, tail)],
                                src_v.at[pl.ds(0, tail)])
                pltpu.sync_copy(ei_hbm.at[1, pl.ds(tail0, tail)],
                                dst_v.at[pl.ds(0, tail)])
                for j in range(tail):
                    one_chunk(j, rows0, sem0)
        plsc.subcore_barrier()

        @pl.when(s < NS - 1)
        def _full():
            pltpu.sync_copy(acc_sh.at[pl.ds(base, pt)],
                            psum_hbm.at[c, pl.ds(base, pt)])

        @pl.when(s == NS - 1)
        def _last():
            pltpu.sync_copy(acc_sh.at[pl.ds(base, lt)],
                            psum_hbm.at[c, pl.ds(base, lt)])

        pltpu.sync_copy(cnt_v.at[pl.ds(0, n)], pcnt_hbm.at[wid, 0])

    return k(x, ei_r)


def _seg_sum_scalars(t, ei_r):
    n = t.shape[0]
    a2 = (n + 1 + L - 1) // L * L
    nch = ei_r.shape[1]
    cb, sl, grem, tail0, tail = _chunk_split(nch)
    mesh = plsc.VectorSubcoreMesh(core_axis_name="c", subcore_axis_name="s")

    @functools.partial(
        pl.kernel,
        out_type=jax.ShapeDtypeStruct((NW, 1, n), jnp.float32),
        mesh=mesh,
        scratch_types=[
            pltpu.VMEM((sl, CK), jnp.int32),
            pltpu.VMEM((sl, CK), jnp.int32),
            pltpu.VMEM((n,), jnp.float32),
            pltpu.VMEM((a2,), jnp.float32),
        ],
        compiler_params=pltpu.CompilerParams(needs_layout_passes=False),
    )
    def k(t_hbm, ei_hbm, out_hbm, src_sl, dst_sl, t_v, acc_v):
        c = lax.axis_index("c")
        s = lax.axis_index("s")
        wid = s * NC + c
        zv = jnp.zeros((L,), jnp.float32)

        def za(i, _):
            acc_v[pl.ds(i * L, L)] = zv
            return 0
        lax.fori_loop(0, a2 // L, za, 0)

        pltpu.sync_copy(t_hbm, t_v)
        lo = wid * cb

        def row(r, _):
            for v in range(CK // L):
                sv = src_sl[r, pl.ds(v * L, L)]
                dv = dst_sl[r, pl.ds(v * L, L)]
                vals = plsc.load_gather(t_v, [sv])
                plsc.addupdate_scatter(acc_v, [dv], vals)
            return 0

        def slab(k2, _):
            off = lo + k2 * sl
            pltpu.sync_copy(ei_hbm.at[0, pl.ds(off, sl)], src_sl)
            pltpu.sync_copy(ei_hbm.at[1, pl.ds(off, sl)], dst_sl)
            lax.fori_loop(0, sl, row, 0)
            return 0
        lax.fori_loop(0, cb // sl, slab, 0)

        if grem:
            @pl.when(wid < grem)
            def _extra():
                xo = NW * cb + wid * 8
                pltpu.sync_copy(ei_hbm.at[0, pl.ds(xo, 8)],
                                src_sl.at[pl.ds(0, 8)])
                pltpu.sync_copy(ei_hbm.at[1, pl.ds(xo, 8)],
                                dst_sl.at[pl.ds(0, 8)])
                lax.fori_loop(0, 8, row, 0)

        if tail:
            @pl.when(wid == grem)
            def _tail():
                pltpu.sync_copy(ei_hbm.at[0, pl.ds(tail0, tail)],
                                src_sl.at[pl.ds(0, tail)])
                pltpu.sync_copy(ei_hbm.at[1, pl.ds(tail0, tail)],
                                dst_sl.at[pl.ds(0, tail)])
                lax.fori_loop(0, tail, row, 0)

        pltpu.sync_copy(acc_v.at[pl.ds(0, n)], out_hbm.at[wid, 0])

    return k(t, ei_r)


def _dense_mid(psum_p, den_p, x_p, wl1k, bl1k, wr1k, wl2k, wr2k, bl2k):
    n8 = x_p.shape[0]

    def body(p_ref, den_ref, x_ref, wl1_ref, bl1_ref, wr1_ref, wl2_ref,
             wr2_ref, bl2_ref, t_ref, r2_ref):
        agg = (p_ref[0] + p_ref[1]) / den_ref[...]
        y = (jnp.dot(agg, wl1_ref[...], preferred_element_type=jnp.float32)
             + bl1_ref[...]
             + jnp.dot(x_ref[...], wr1_ref[...],
                       preferred_element_type=jnp.float32))
        y = jnp.maximum(y, 0.0)
        t_ref[...] = jnp.dot(y, wl2_ref[...],
                             preferred_element_type=jnp.float32)
        r2_ref[...] = (jnp.dot(y, wr2_ref[...],
                               preferred_element_type=jnp.float32)
                       + bl2_ref[...])

    return pl.pallas_call(
        body,
        out_shape=[
            jax.ShapeDtypeStruct((n8, 8), jnp.float32),
            jax.ShapeDtypeStruct((n8, 8), jnp.float32),
        ],
    )(psum_p, den_p, x_p, wl1k, bl1k, wr1k, wl2k, wr2k, bl2k)


def _dense_out(q2, pcnt2, r2row):
    n = q2.shape[1]

    def body(q_ref, c_ref, r2_ref, o_ref):
        q = jnp.sum(q_ref[...], axis=0, keepdims=True)
        den = jnp.maximum(jnp.sum(c_ref[...], axis=0), 1.0)
        o_ref[...] = q / den + r2_ref[...]

    return pl.pallas_call(
        body,
        out_shape=jax.ShapeDtypeStruct((1, n), jnp.float32),
    )(q2, pcnt2, r2row)


def kernel(edge_index, user_emb, movie_emb, W_l1, b_l1, W_r1, W_l2, b_l2,
           W_r2):
    x = jnp.concatenate([user_emb, movie_emb], axis=0)
    n = x.shape[0]
    e = edge_index.shape[1]

    ei = edge_index.astype(jnp.int32)
    if e % CK:
        pad = CK - e % CK
        ei = jnp.concatenate(
            [ei, jnp.stack([jnp.zeros((pad,), jnp.int32),
                            jnp.full((pad,), n, jnp.int32)])], axis=1)
    ei_r = ei.reshape(2, -1, CK)

    np_rows = (n + 1 + NS * L - 1) // (NS * L) * (NS * L)

    psum, pcnt = _seg_sum_rows(x, ei_r, np_rows)

    eye = jnp.eye(8, dtype=jnp.float32)
    denrow = jnp.maximum(jnp.sum(pcnt[:, 0, :], axis=0), 1.0)
    den_p = jnp.broadcast_to(denrow[:, None], (n, D)).reshape(n // 8, 8 * D)
    t8, r28 = _dense_mid(
        psum.reshape(NC, n // 8, 8 * D), den_p, x.reshape(n // 8, 8 * D),
        jnp.kron(eye, W_l1), jnp.tile(b_l1, 8).reshape(1, 8 * D),
        jnp.kron(eye, W_r1), jnp.kron(eye, W_l2), jnp.kron(eye, W_r2),
        jnp.broadcast_to(b_l2.reshape(1, 1), (1, 8)))

    q = _seg_sum_scalars(t8.reshape(n), ei_r).reshape(NW, n)
    out = _dense_out(q, pcnt, r28.reshape(1, n))
    return out.reshape(n, 1)


# speedup vs baseline: 46.8691x; 1.1063x over previous
import functools

import jax
import jax.numpy as jnp
from jax import lax
from jax.experimental import pallas as pl
from jax.experimental.pallas import tpu as pltpu
from jax.experimental.pallas import tpu_sc as plsc

D = 16
L = 16
NC = 2
NS = 16
NW = NC * NS
CK = 128


def _chunk_split(nch):
    g8 = nch // 8
    gb = g8 // NW
    cb = gb * 8
    grem = g8 % NW
    tail0 = NW * cb + grem * 8
    tail = nch - tail0
    sl = 8
    for cand in range(56, 7, -8):
        if cb % cand == 0:
            sl = cand
            break
    return cb, sl, grem, tail0, tail


def _seg_sum_rows(x, ei_r, np_rows):
    n = x.shape[0]
    a2 = (n + 1 + L - 1) // L * L
    nch = ei_r.shape[1]
    cb, sl, grem, tail0, tail = _chunk_split(nch)
    pt = np_rows // NS
    lt = n - (NS - 1) * pt
    zr = pt // 8
    mesh = plsc.VectorSubcoreMesh(core_axis_name="c", subcore_axis_name="s")

    @functools.partial(
        pl.kernel,
        out_type=(
            jax.ShapeDtypeStruct((NC, n, D), jnp.float32),
            jax.ShapeDtypeStruct((NW, 1, n), jnp.float32),
        ),
        mesh=mesh,
        scratch_types=[
            pltpu.VMEM((sl, CK), jnp.int32),
            pltpu.VMEM((sl, CK), jnp.int32),
            pltpu.VMEM((CK, D), jnp.float32),
            pltpu.VMEM((CK, D), jnp.float32),
            pltpu.VMEM((CK, D), jnp.float32),
            pltpu.VMEM((CK, D), jnp.float32),
            pltpu.VMEM((zr, D), jnp.float32),
            pltpu.VMEM((a2,), jnp.float32),
            pltpu.VMEM_SHARED((np_rows, D), jnp.float32),
            pltpu.SemaphoreType.DMA,
            pltpu.SemaphoreType.DMA,
            pltpu.SemaphoreType.DMA,
            pltpu.SemaphoreType.DMA,
            pltpu.SemaphoreType.DMA,
        ],
        compiler_params=pltpu.CompilerParams(use_tc_tiling_on_sc=False,
                                             needs_layout_passes=False),
    )
    def k(x_hbm, ei_hbm, psum_hbm, pcnt_hbm, src_v, dst_v, rows0, rows1,
          rows2, rows3, zacc_v, cnt_v, acc_sh, sem0, sem1, sem2, sem3,
          sem4):
        c = lax.axis_index("c")
        s = lax.axis_index("s")
        wid = s * NC + c
        base = s * pt
        zv = jnp.zeros((L,), jnp.float32)
        ov = jnp.ones((L,), jnp.float32)
        rows = (rows0, rows1, rows2, rows3)
        sems = (sem0, sem1, sem2, sem3)

        def zb(i, _):
            for u in range(8):
                zacc_v[i * 8 + u] = zv
            return 0
        lax.fori_loop(0, zr // 8, zb, 0)
        for u in range(zr % 8):
            zacc_v[zr - zr % 8 + u] = zv

        def zc(i, _):
            for u in range(8):
                cnt_v[pl.ds((i * 8 + u) * L, L)] = zv
            return 0
        lax.fori_loop(0, a2 // L // 8, zc, 0)
        for u in range(a2 // L % 8):
            cnt_v[pl.ds((a2 // L - a2 // L % 8 + u) * L, L)] = zv

        for j in range(8):
            pltpu.sync_copy(zacc_v, acc_sh.at[pl.ds(base + j * zr, zr)])
        plsc.subcore_barrier()

        lo = wid * cb

        def one_chunk(j, rbuf, sem):
            pltpu.async_copy(x_hbm.at[src_v.at[j]], rbuf, sem).wait()
            pltpu.sync_copy(rbuf, acc_sh.at[dst_v.at[j]], add=True)
            for v in range(CK // L):
                dv = dst_v[j, pl.ds(v * L, L)]
                plsc.addupdate_scatter(cnt_v, [dv], ov)

        def quad(qq, _):
            j = qq * 4
            gds = [pltpu.async_copy(x_hbm.at[src_v.at[j + b]], rows[b],
                                    sems[b]) for b in range(4)]
            for b in range(4):
                for v in range(CK // L):
                    dv = dst_v[j + b, pl.ds(v * L, L)]
                    plsc.addupdate_scatter(cnt_v, [dv], ov)
            for b in range(4):
                gds[b].wait()
                pltpu.sync_copy(rows[b], acc_sh.at[dst_v.at[j + b]],
                                add=True)
            return 0

        def slab(k2, _):
            off = lo + k2 * sl
            pltpu.sync_copy(ei_hbm.at[0, pl.ds(off, sl)], src_v)
            pltpu.sync_copy(ei_hbm.at[1, pl.ds(off, sl)], dst_v)
            lax.fori_loop(0, sl // 4, quad, 0)
            return 0
        lax.fori_loop(0, cb // sl, slab, 0)

        if grem:
            @pl.when(wid < grem)
            def _extra():
                xo = NW * cb + wid * 8
                pltpu.sync_copy(ei_hbm.at[0, pl.ds(xo, 8)],
                                src_v.at[pl.ds(0, 8)])
                pltpu.sync_copy(ei_hbm.at[1, pl.ds(xo, 8)],
                                dst_v.at[pl.ds(0, 8)])
                lax.fori_loop(0, 2, quad, 0)

        if tail:
            @pl.when(wid == grem)
            def _tail():
                pltpu.sync_copy(ei_hbm.at[0, pl.ds(tail0, tail)],
                                src_v.at[pl.ds(0, tail)])
                pltpu.sync_copy(ei_hbm.at[1, pl.ds(tail0, tail)],
                                dst_v.at[pl.ds(0, tail)])
                for j in range(tail):
                    one_chunk(j, rows0, sem0)
        plsc.subcore_barrier()

        @pl.when(s < NS - 1)
        def _full():
            pltpu.sync_copy(acc_sh.at[pl.ds(base, pt)],
                            psum_hbm.at[c, pl.ds(base, pt)])

        @pl.when(s == NS - 1)
        def _last():
            pltpu.sync_copy(acc_sh.at[pl.ds(base, lt)],
                            psum_hbm.at[c, pl.ds(base, lt)])

        pltpu.sync_copy(cnt_v.at[pl.ds(0, n)], pcnt_hbm.at[wid, 0])

    return k(x, ei_r)


def _seg_sum_scalars(t, ei_r):
    n = t.shape[0]
    a2 = (n + 1 + L - 1) // L * L
    nch = ei_r.shape[1]
    cb, sl, grem, tail0, tail = _chunk_split(nch)
    mesh = plsc.VectorSubcoreMesh(core_axis_name="c", subcore_axis_name="s")

    @functools.partial(
        pl.kernel,
        out_type=jax.ShapeDtypeStruct((NW, 1, n), jnp.float32),
        mesh=mesh,
        scratch_types=[
            pltpu.VMEM((sl, CK), jnp.int32),
            pltpu.VMEM((sl, CK), jnp.int32),
            pltpu.VMEM((n,), jnp.float32),
            pltpu.VMEM((a2,), jnp.float32),
        ],
        compiler_params=pltpu.CompilerParams(needs_layout_passes=False),
    )
    def k(t_hbm, ei_hbm, out_hbm, src_sl, dst_sl, t_v, acc_v):
        c = lax.axis_index("c")
        s = lax.axis_index("s")
        wid = s * NC + c
        zv = jnp.zeros((L,), jnp.float32)

        def za(i, _):
            for u in range(8):
                acc_v[pl.ds((i * 8 + u) * L, L)] = zv
            return 0
        lax.fori_loop(0, a2 // L // 8, za, 0)
        for u in range(a2 // L % 8):
            acc_v[pl.ds((a2 // L - a2 // L % 8 + u) * L, L)] = zv

        pltpu.sync_copy(t_hbm, t_v)
        lo = wid * cb

        def row(r, _):
            for v in range(CK // L):
                sv = src_sl[r, pl.ds(v * L, L)]
                dv = dst_sl[r, pl.ds(v * L, L)]
                vals = plsc.load_gather(t_v, [sv])
                plsc.addupdate_scatter(acc_v, [dv], vals)
            return 0

        def slab(k2, _):
            off = lo + k2 * sl
            pltpu.sync_copy(ei_hbm.at[0, pl.ds(off, sl)], src_sl)
            pltpu.sync_copy(ei_hbm.at[1, pl.ds(off, sl)], dst_sl)
            lax.fori_loop(0, sl, row, 0)
            return 0
        lax.fori_loop(0, cb // sl, slab, 0)

        if grem:
            @pl.when(wid < grem)
            def _extra():
                xo = NW * cb + wid * 8
                pltpu.sync_copy(ei_hbm.at[0, pl.ds(xo, 8)],
                                src_sl.at[pl.ds(0, 8)])
                pltpu.sync_copy(ei_hbm.at[1, pl.ds(xo, 8)],
                                dst_sl.at[pl.ds(0, 8)])
                lax.fori_loop(0, 8, row, 0)

        if tail:
            @pl.when(wid == grem)
            def _tail():
                pltpu.sync_copy(ei_hbm.at[0, pl.ds(tail0, tail)],
                                src_sl.at[pl.ds(0, tail)])
                pltpu.sync_copy(ei_hbm.at[1, pl.ds(tail0, tail)],
                                dst_sl.at[pl.ds(0, tail)])
                lax.fori_loop(0, tail, row, 0)

        pltpu.sync_copy(acc_v.at[pl.ds(0, n)], out_hbm.at[wid, 0])

    return k(t, ei_r)


def _dense_mid(psum_p, den_p, x_p, wl1k, bl1k, wr1k, wl2k, wr2k, bl2k):
    n8 = x_p.shape[0]

    def body(p_ref, den_ref, x_ref, wl1_ref, bl1_ref, wr1_ref, wl2_ref,
             wr2_ref, bl2_ref, t_ref, r2_ref):
        agg = (p_ref[0] + p_ref[1]) / den_ref[...]
        y = (jnp.dot(agg, wl1_ref[...], preferred_element_type=jnp.float32)
             + bl1_ref[...]
             + jnp.dot(x_ref[...], wr1_ref[...],
                       preferred_element_type=jnp.float32))
        y = jnp.maximum(y, 0.0)
        t_ref[...] = jnp.dot(y, wl2_ref[...],
                             preferred_element_type=jnp.float32)
        r2_ref[...] = (jnp.dot(y, wr2_ref[...],
                               preferred_element_type=jnp.float32)
                       + bl2_ref[...])

    return pl.pallas_call(
        body,
        out_shape=[
            jax.ShapeDtypeStruct((n8, 8), jnp.float32),
            jax.ShapeDtypeStruct((n8, 8), jnp.float32),
        ],
    )(psum_p, den_p, x_p, wl1k, bl1k, wr1k, wl2k, wr2k, bl2k)


def _dense_out(q2, pcnt2, r2row):
    n = q2.shape[1]

    def body(q_ref, c_ref, r2_ref, o_ref):
        q = jnp.sum(q_ref[...], axis=0, keepdims=True)
        den = jnp.maximum(jnp.sum(c_ref[...], axis=0), 1.0)
        o_ref[...] = q / den + r2_ref[...]

    return pl.pallas_call(
        body,
        out_shape=jax.ShapeDtypeStruct((1, n), jnp.float32),
    )(q2, pcnt2, r2row)


def kernel(edge_index, user_emb, movie_emb, W_l1, b_l1, W_r1, W_l2, b_l2,
           W_r2):
    x = jnp.concatenate([user_emb, movie_emb], axis=0)
    n = x.shape[0]
    e = edge_index.shape[1]

    ei = edge_index.astype(jnp.int32)
    if e % CK:
        pad = CK - e % CK
        ei = jnp.concatenate(
            [ei, jnp.stack([jnp.zeros((pad,), jnp.int32),
                            jnp.full((pad,), n, jnp.int32)])], axis=1)
    ei_r = ei.reshape(2, -1, CK)

    np_rows = (n + 1 + NS * L - 1) // (NS * L) * (NS * L)

    psum, pcnt = _seg_sum_rows(x, ei_r, np_rows)

    eye = jnp.eye(8, dtype=jnp.float32)
    denrow = jnp.maximum(jnp.sum(pcnt[:, 0, :], axis=0), 1.0)
    den_p = jnp.broadcast_to(denrow[:, None], (n, D)).reshape(n // 8, 8 * D)
    t8, r28 = _dense_mid(
        psum.reshape(NC, n // 8, 8 * D), den_p, x.reshape(n // 8, 8 * D),
        jnp.kron(eye, W_l1), jnp.tile(b_l1, 8).reshape(1, 8 * D),
        jnp.kron(eye, W_r1), jnp.kron(eye, W_l2), jnp.kron(eye, W_r2),
        jnp.broadcast_to(b_l2.reshape(1, 1), (1, 8)))

    q = _seg_sum_scalars(t8.reshape(n), ei_r).reshape(NW, n)
    out = _dense_out(q, pcnt, r28.reshape(1, n))
    return out.reshape(n, 1)


# speedup vs baseline: 49.0688x; 1.0469x over previous
import functools

import jax
import jax.numpy as jnp
from jax import lax
from jax.experimental import pallas as pl
from jax.experimental.pallas import tpu as pltpu
from jax.experimental.pallas import tpu_sc as plsc

D = 16
L = 16
NC = 2
NS = 16
NW = NC * NS
CK = 128


def _chunk_split(nch):
    g8 = nch // 8
    gb = g8 // NW
    cb = gb * 8
    grem = g8 % NW
    tail0 = NW * cb + grem * 8
    tail = nch - tail0
    sl = 8
    for cand in range(56, 7, -8):
        if cb % cand == 0:
            sl = cand
            break
    return cb, sl, grem, tail0, tail


def _seg_sum_rows(x, ei_r, np_rows):
    n = x.shape[0]
    a2 = (n + 1 + L - 1) // L * L
    nch = ei_r.shape[1]
    cb, sl, grem, tail0, tail = _chunk_split(nch)
    pt = np_rows // NS
    lt = n - (NS - 1) * pt
    zr = pt // 8
    mesh = plsc.VectorSubcoreMesh(core_axis_name="c", subcore_axis_name="s")

    @functools.partial(
        pl.kernel,
        out_type=(
            jax.ShapeDtypeStruct((NC, n, D), jnp.float32),
            jax.ShapeDtypeStruct((NW, 1, n), jnp.float32),
        ),
        mesh=mesh,
        scratch_types=[
            pltpu.VMEM((sl, CK), jnp.int32),
            pltpu.VMEM((sl, CK), jnp.int32),
            pltpu.VMEM((CK, D), jnp.float32),
            pltpu.VMEM((CK, D), jnp.float32),
            pltpu.VMEM((CK, D), jnp.float32),
            pltpu.VMEM((CK, D), jnp.float32),
            pltpu.VMEM((zr, D), jnp.float32),
            pltpu.VMEM((a2,), jnp.float32),
            pltpu.VMEM_SHARED((np_rows, D), jnp.float32),
            pltpu.SemaphoreType.DMA,
            pltpu.SemaphoreType.DMA,
            pltpu.SemaphoreType.DMA,
            pltpu.SemaphoreType.DMA,
            pltpu.SemaphoreType.DMA,
            pltpu.SemaphoreType.DMA,
            pltpu.SemaphoreType.DMA,
            pltpu.SemaphoreType.DMA,
        ],
        compiler_params=pltpu.CompilerParams(use_tc_tiling_on_sc=False,
                                             needs_layout_passes=False),
    )
    def k(x_hbm, ei_hbm, psum_hbm, pcnt_hbm, src_v, dst_v, rows0, rows1,
          rows2, rows3, zacc_v, cnt_v, acc_sh, sem0, sem1, sem2, sem3,
          sem4, sem5, sem6, sem7):
        c = lax.axis_index("c")
        s = lax.axis_index("s")
        wid = s * NC + c
        base = s * pt
        zv = jnp.zeros((L,), jnp.float32)
        ov = jnp.ones((L,), jnp.float32)
        rows = (rows0, rows1, rows2, rows3)
        sems = (sem0, sem1, sem2, sem3)
        ssems = (sem4, sem5, sem6, sem7)

        def zb(i, _):
            for u in range(8):
                zacc_v[i * 8 + u] = zv
            return 0
        lax.fori_loop(0, zr // 8, zb, 0)
        for u in range(zr % 8):
            zacc_v[zr - zr % 8 + u] = zv

        def zc(i, _):
            for u in range(8):
                cnt_v[pl.ds((i * 8 + u) * L, L)] = zv
            return 0
        lax.fori_loop(0, a2 // L // 8, zc, 0)
        for u in range(a2 // L % 8):
            cnt_v[pl.ds((a2 // L - a2 // L % 8 + u) * L, L)] = zv

        for j in range(8):
            pltpu.sync_copy(zacc_v, acc_sh.at[pl.ds(base + j * zr, zr)])
        plsc.subcore_barrier()

        lo = wid * cb

        def one_chunk(j, rbuf, sem):
            pltpu.async_copy(x_hbm.at[src_v.at[j]], rbuf, sem).wait()
            pltpu.sync_copy(rbuf, acc_sh.at[dst_v.at[j]], add=True)
            for v in range(CK // L):
                dv = dst_v[j, pl.ds(v * L, L)]
                plsc.addupdate_scatter(cnt_v, [dv], ov)

        def quad(qq, _):
            j = qq * 4
            gds = [pltpu.async_copy(x_hbm.at[src_v.at[j + b]], rows[b],
                                    sems[b]) for b in range(4)]
            for b in range(4):
                for v in range(CK // L):
                    dv = dst_v[j + b, pl.ds(v * L, L)]
                    plsc.addupdate_scatter(cnt_v, [dv], ov)
            sds = []
            for b in range(4):
                gds[b].wait()
                sds.append(pltpu.async_copy(
                    rows[b], acc_sh.at[dst_v.at[j + b]], ssems[b],
                    add=True))
            for d in sds:
                d.wait()
            return 0

        def slab(k2, _):
            off = lo + k2 * sl
            pltpu.sync_copy(ei_hbm.at[0, pl.ds(off, sl)], src_v)
            pltpu.sync_copy(ei_hbm.at[1, pl.ds(off, sl)], dst_v)
            lax.fori_loop(0, sl // 4, quad, 0)
            return 0
        lax.fori_loop(0, cb // sl, slab, 0)

        if grem:
            @pl.when(wid < grem)
            def _extra():
                xo = NW * cb + wid * 8
                pltpu.sync_copy(ei_hbm.at[0, pl.ds(xo, 8)],
                                src_v.at[pl.ds(0, 8)])
                pltpu.sync_copy(ei_hbm.at[1, pl.ds(xo, 8)],
                                dst_v.at[pl.ds(0, 8)])
                lax.fori_loop(0, 2, quad, 0)

        if tail:
            @pl.when(wid == grem)
            def _tail():
                pltpu.sync_copy(ei_hbm.at[0, pl.ds(tail0, tail)],
                                src_v.at[pl.ds(0, tail)])
                pltpu.sync_copy(ei_hbm.at[1, pl.ds(tail0, tail)],
                                dst_v.at[pl.ds(0, tail)])
                for j in range(tail):
                    one_chunk(j, rows0, sem0)
        plsc.subcore_barrier()

        @pl.when(s < NS - 1)
        def _full():
            pltpu.sync_copy(acc_sh.at[pl.ds(base, pt)],
                            psum_hbm.at[c, pl.ds(base, pt)])

        @pl.when(s == NS - 1)
        def _last():
            pltpu.sync_copy(acc_sh.at[pl.ds(base, lt)],
                            psum_hbm.at[c, pl.ds(base, lt)])

        pltpu.sync_copy(cnt_v.at[pl.ds(0, n)], pcnt_hbm.at[wid, 0])

    return k(x, ei_r)


def _seg_sum_scalars(t, ei_r):
    n = t.shape[0]
    a2 = (n + 1 + L - 1) // L * L
    nch = ei_r.shape[1]
    cb, sl, grem, tail0, tail = _chunk_split(nch)
    mesh = plsc.VectorSubcoreMesh(core_axis_name="c", subcore_axis_name="s")

    @functools.partial(
        pl.kernel,
        out_type=jax.ShapeDtypeStruct((NW, 1, n), jnp.float32),
        mesh=mesh,
        scratch_types=[
            pltpu.VMEM((sl, CK), jnp.int32),
            pltpu.VMEM((sl, CK), jnp.int32),
            pltpu.VMEM((n,), jnp.float32),
            pltpu.VMEM((a2,), jnp.float32),
        ],
        compiler_params=pltpu.CompilerParams(needs_layout_passes=False),
    )
    def k(t_hbm, ei_hbm, out_hbm, src_sl, dst_sl, t_v, acc_v):
        c = lax.axis_index("c")
        s = lax.axis_index("s")
        wid = s * NC + c
        zv = jnp.zeros((L,), jnp.float32)

        def za(i, _):
            for u in range(8):
                acc_v[pl.ds((i * 8 + u) * L, L)] = zv
            return 0
        lax.fori_loop(0, a2 // L // 8, za, 0)
        for u in range(a2 // L % 8):
            acc_v[pl.ds((a2 // L - a2 // L % 8 + u) * L, L)] = zv

        pltpu.sync_copy(t_hbm, t_v)
        lo = wid * cb

        def row(r, _):
            for v in range(CK // L):
                sv = src_sl[r, pl.ds(v * L, L)]
                dv = dst_sl[r, pl.ds(v * L, L)]
                vals = plsc.load_gather(t_v, [sv])
                plsc.addupdate_scatter(acc_v, [dv], vals)
            return 0

        def slab(k2, _):
            off = lo + k2 * sl
            pltpu.sync_copy(ei_hbm.at[0, pl.ds(off, sl)], src_sl)
            pltpu.sync_copy(ei_hbm.at[1, pl.ds(off, sl)], dst_sl)
            lax.fori_loop(0, sl, row, 0)
            return 0
        lax.fori_loop(0, cb // sl, slab, 0)

        if grem:
            @pl.when(wid < grem)
            def _extra():
                xo = NW * cb + wid * 8
                pltpu.sync_copy(ei_hbm.at[0, pl.ds(xo, 8)],
                                src_sl.at[pl.ds(0, 8)])
                pltpu.sync_copy(ei_hbm.at[1, pl.ds(xo, 8)],
                                dst_sl.at[pl.ds(0, 8)])
                lax.fori_loop(0, 8, row, 0)

        if tail:
            @pl.when(wid == grem)
            def _tail():
                pltpu.sync_copy(ei_hbm.at[0, pl.ds(tail0, tail)],
                                src_sl.at[pl.ds(0, tail)])
                pltpu.sync_copy(ei_hbm.at[1, pl.ds(tail0, tail)],
                                dst_sl.at[pl.ds(0, tail)])
                lax.fori_loop(0, tail, row, 0)

        pltpu.sync_copy(acc_v.at[pl.ds(0, n)], out_hbm.at[wid, 0])

    return k(t, ei_r)


def _dense_mid(psum_p, den_p, x_p, wl1k, bl1k, wr1k, wl2k, wr2k, bl2k):
    n8 = x_p.shape[0]

    def body(p_ref, den_ref, x_ref, wl1_ref, bl1_ref, wr1_ref, wl2_ref,
             wr2_ref, bl2_ref, t_ref, r2_ref):
        agg = (p_ref[0] + p_ref[1]) / den_ref[...]
        y = (jnp.dot(agg, wl1_ref[...], preferred_element_type=jnp.float32)
             + bl1_ref[...]
             + jnp.dot(x_ref[...], wr1_ref[...],
                       preferred_element_type=jnp.float32))
        y = jnp.maximum(y, 0.0)
        t_ref[...] = jnp.dot(y, wl2_ref[...],
                             preferred_element_type=jnp.float32)
        r2_ref[...] = (jnp.dot(y, wr2_ref[...],
                               preferred_element_type=jnp.float32)
                       + bl2_ref[...])

    return pl.pallas_call(
        body,
        out_shape=[
            jax.ShapeDtypeStruct((n8, 8), jnp.float32),
            jax.ShapeDtypeStruct((n8, 8), jnp.float32),
        ],
    )(psum_p, den_p, x_p, wl1k, bl1k, wr1k, wl2k, wr2k, bl2k)


def _dense_out(q2, pcnt2, r2row):
    n = q2.shape[1]

    def body(q_ref, c_ref, r2_ref, o_ref):
        q = jnp.sum(q_ref[...], axis=0, keepdims=True)
        den = jnp.maximum(jnp.sum(c_ref[...], axis=0), 1.0)
        o_ref[...] = q / den + r2_ref[...]

    return pl.pallas_call(
        body,
        out_shape=jax.ShapeDtypeStruct((1, n), jnp.float32),
    )(q2, pcnt2, r2row)


def kernel(edge_index, user_emb, movie_emb, W_l1, b_l1, W_r1, W_l2, b_l2,
           W_r2):
    x = jnp.concatenate([user_emb, movie_emb], axis=0)
    n = x.shape[0]
    e = edge_index.shape[1]

    ei = edge_index.astype(jnp.int32)
    if e % CK:
        pad = CK - e % CK
        ei = jnp.concatenate(
            [ei, jnp.stack([jnp.zeros((pad,), jnp.int32),
                            jnp.full((pad,), n, jnp.int32)])], axis=1)
    ei_r = ei.reshape(2, -1, CK)

    np_rows = (n + 1 + NS * L - 1) // (NS * L) * (NS * L)

    psum, pcnt = _seg_sum_rows(x, ei_r, np_rows)

    eye = jnp.eye(8, dtype=jnp.float32)
    denrow = jnp.maximum(jnp.sum(pcnt[:, 0, :], axis=0), 1.0)
    den_p = jnp.broadcast_to(denrow[:, None], (n, D)).reshape(n // 8, 8 * D)
    t8, r28 = _dense_mid(
        psum.reshape(NC, n // 8, 8 * D), den_p, x.reshape(n // 8, 8 * D),
        jnp.kron(eye, W_l1), jnp.tile(b_l1, 8).reshape(1, 8 * D),
        jnp.kron(eye, W_r1), jnp.kron(eye, W_l2), jnp.kron(eye, W_r2),
        jnp.broadcast_to(b_l2.reshape(1, 1), (1, 8)))

    q = _seg_sum_scalars(t8.reshape(n), ei_r).reshape(NW, n)
    out = _dense_out(q, pcnt, r28.reshape(1, n))
    return out.reshape(n, 1)


# speedup vs baseline: 50.8753x; 1.0368x over previous
import functools

import jax
import jax.numpy as jnp
from jax import lax
from jax.experimental import pallas as pl
from jax.experimental.pallas import tpu as pltpu
from jax.experimental.pallas import tpu_sc as plsc

D = 16
L = 16
NC = 2
NS = 16
NW = NC * NS
CK = 128


def _chunk_split(nch):
    g8 = nch // 8
    gb = g8 // NW
    cb = gb * 8
    grem = g8 % NW
    tail0 = NW * cb + grem * 8
    tail = nch - tail0
    sl = 8
    for cand in range(56, 7, -8):
        if cb % cand == 0:
            sl = cand
            break
    return cb, sl, grem, tail0, tail


def _seg_sum_rows(x, ei_r, np_rows):
    n = x.shape[0]
    a2 = (n + 1 + L - 1) // L * L
    nch = ei_r.shape[1]
    cb, sl, grem, tail0, tail = _chunk_split(nch)
    pt = np_rows // NS
    lt = n - (NS - 1) * pt
    zr = pt // 8
    mesh = plsc.VectorSubcoreMesh(core_axis_name="c", subcore_axis_name="s")

    sl1 = sl // 2
    dbuf = (sl1 % 8 == 0 and sl1 % 4 == 0 and cb % sl1 == 0
            and (cb // sl1) % 2 == 0 and NW * cb + sl1 <= nch)
    if not dbuf:
        sl1 = sl

    @functools.partial(
        pl.kernel,
        out_type=(
            jax.ShapeDtypeStruct((NC, n, D), jnp.float32),
            jax.ShapeDtypeStruct((NW, 1, n), jnp.float32),
        ),
        mesh=mesh,
        scratch_types=[
            pltpu.VMEM((sl1, CK), jnp.int32),
            pltpu.VMEM((sl1, CK), jnp.int32),
            pltpu.VMEM((sl1, CK), jnp.int32),
            pltpu.VMEM((sl1, CK), jnp.int32),
            pltpu.VMEM((CK, D), jnp.float32),
            pltpu.VMEM((CK, D), jnp.float32),
            pltpu.VMEM((CK, D), jnp.float32),
            pltpu.VMEM((CK, D), jnp.float32),
            pltpu.VMEM((zr, D), jnp.float32),
            pltpu.VMEM((a2,), jnp.float32),
            pltpu.VMEM_SHARED((np_rows, D), jnp.float32),
        ] + [pltpu.SemaphoreType.DMA] * 12,
        compiler_params=pltpu.CompilerParams(use_tc_tiling_on_sc=False,
                                             needs_layout_passes=False),
    )
    def k(x_hbm, ei_hbm, psum_hbm, pcnt_hbm, src_v, dst_v, src_v2, dst_v2,
          rows0, rows1, rows2, rows3, zacc_v, cnt_v, acc_sh, sem0, sem1,
          sem2, sem3, sem4, sem5, sem6, sem7, lsa0, lsd0, lsa1, lsd1):
        c = lax.axis_index("c")
        s = lax.axis_index("s")
        wid = s * NC + c
        base = s * pt
        zv = jnp.zeros((L,), jnp.float32)
        ov = jnp.ones((L,), jnp.float32)
        rows = (rows0, rows1, rows2, rows3)
        sems = (sem0, sem1, sem2, sem3)
        ssems = (sem4, sem5, sem6, sem7)

        def zb(i, _):
            for u in range(8):
                zacc_v[i * 8 + u] = zv
            return 0
        lax.fori_loop(0, zr // 8, zb, 0)
        for u in range(zr % 8):
            zacc_v[zr - zr % 8 + u] = zv

        def zc(i, _):
            for u in range(8):
                cnt_v[pl.ds((i * 8 + u) * L, L)] = zv
            return 0
        lax.fori_loop(0, a2 // L // 8, zc, 0)
        for u in range(a2 // L % 8):
            cnt_v[pl.ds((a2 // L - a2 // L % 8 + u) * L, L)] = zv

        for j in range(8):
            pltpu.sync_copy(zacc_v, acc_sh.at[pl.ds(base + j * zr, zr)])
        plsc.subcore_barrier()

        lo = wid * cb

        def one_chunk(j, rbuf, sem):
            pltpu.async_copy(x_hbm.at[src_v.at[j]], rbuf, sem).wait()
            pltpu.sync_copy(rbuf, acc_sh.at[dst_v.at[j]], add=True)
            for v in range(CK // L):
                dv = dst_v[j, pl.ds(v * L, L)]
                plsc.addupdate_scatter(cnt_v, [dv], ov)

        def make_quad(sv, dv_ref):
            def quad(qq, _):
                j = qq * 4
                gds = [pltpu.async_copy(x_hbm.at[sv.at[j + b]], rows[b],
                                        sems[b]) for b in range(4)]
                for b in range(4):
                    for v in range(CK // L):
                        dv = dv_ref[j + b, pl.ds(v * L, L)]
                        plsc.addupdate_scatter(cnt_v, [dv], ov)
                sds = []
                for b in range(4):
                    gds[b].wait()
                    sds.append(pltpu.async_copy(
                        rows[b], acc_sh.at[dv_ref.at[j + b]], ssems[b],
                        add=True))
                for d in sds:
                    d.wait()
                return 0
            return quad

        quad = make_quad(src_v, dst_v)

        banks = ((src_v, dst_v, lsa0, lsd0), (src_v2, dst_v2, lsa1, lsd1))

        def fire_slab(k2, bank):
            sv, dv, sa, sd = banks[bank]
            off = lo + k2 * sl1
            pltpu.async_copy(ei_hbm.at[0, pl.ds(off, sl1)], sv, sa)
            pltpu.async_copy(ei_hbm.at[1, pl.ds(off, sl1)], dv, sd)

        def wait_slab(bank):
            sv, dv, sa, sd = banks[bank]
            pltpu.make_async_copy(ei_hbm.at[0, pl.ds(0, sl1)], sv, sa).wait()
            pltpu.make_async_copy(ei_hbm.at[1, pl.ds(0, sl1)], dv, sd).wait()

        if dbuf:
            quad_b = make_quad(src_v2, dst_v2)
            fire_slab(0, 0)

            def dslab(q, _):
                fire_slab(2 * q + 1, 1)
                wait_slab(0)
                lax.fori_loop(0, sl1 // 4, quad, 0)
                fire_slab(2 * q + 2, 0)
                wait_slab(1)
                lax.fori_loop(0, sl1 // 4, quad_b, 0)
                return 0
            lax.fori_loop(0, cb // sl1 // 2, dslab, 0)
            wait_slab(0)
        else:
            def slab(k2, _):
                off = lo + k2 * sl1
                pltpu.sync_copy(ei_hbm.at[0, pl.ds(off, sl1)], src_v)
                pltpu.sync_copy(ei_hbm.at[1, pl.ds(off, sl1)], dst_v)
                lax.fori_loop(0, sl1 // 4, quad, 0)
                return 0
            lax.fori_loop(0, cb // sl1, slab, 0)

        if grem:
            @pl.when(wid < grem)
            def _extra():
                xo = NW * cb + wid * 8
                pltpu.sync_copy(ei_hbm.at[0, pl.ds(xo, 8)],
                                src_v.at[pl.ds(0, 8)])
                pltpu.sync_copy(ei_hbm.at[1, pl.ds(xo, 8)],
                                dst_v.at[pl.ds(0, 8)])
                lax.fori_loop(0, 2, quad, 0)

        if tail:
            @pl.when(wid == grem)
            def _tail():
                pltpu.sync_copy(ei_hbm.at[0, pl.ds(tail0, tail)],
                                src_v.at[pl.ds(0, tail)])
                pltpu.sync_copy(ei_hbm.at[1, pl.ds(tail0, tail)],
                                dst_v.at[pl.ds(0, tail)])
                for j in range(tail):
                    one_chunk(j, rows0, sem0)
        plsc.subcore_barrier()

        @pl.when(s < NS - 1)
        def _full():
            pltpu.sync_copy(acc_sh.at[pl.ds(base, pt)],
                            psum_hbm.at[c, pl.ds(base, pt)])

        @pl.when(s == NS - 1)
        def _last():
            pltpu.sync_copy(acc_sh.at[pl.ds(base, lt)],
                            psum_hbm.at[c, pl.ds(base, lt)])

        pltpu.sync_copy(cnt_v.at[pl.ds(0, n)], pcnt_hbm.at[wid, 0])

    return k(x, ei_r)


def _seg_sum_scalars(t, ei_r):
    n = t.shape[0]
    a2 = (n + 1 + L - 1) // L * L
    nch = ei_r.shape[1]
    cb, sl, grem, tail0, tail = _chunk_split(nch)
    mesh = plsc.VectorSubcoreMesh(core_axis_name="c", subcore_axis_name="s")

    @functools.partial(
        pl.kernel,
        out_type=jax.ShapeDtypeStruct((NW, 1, n), jnp.float32),
        mesh=mesh,
        scratch_types=[
            pltpu.VMEM((sl, CK), jnp.int32),
            pltpu.VMEM((sl, CK), jnp.int32),
            pltpu.VMEM((n,), jnp.float32),
            pltpu.VMEM((a2,), jnp.float32),
            pltpu.SemaphoreType.DMA,
        ],
        compiler_params=pltpu.CompilerParams(needs_layout_passes=False),
    )
    def k(t_hbm, ei_hbm, out_hbm, src_sl, dst_sl, t_v, acc_v, semt):
        c = lax.axis_index("c")
        s = lax.axis_index("s")
        wid = s * NC + c
        zv = jnp.zeros((L,), jnp.float32)

        td = pltpu.async_copy(t_hbm, t_v, semt)

        def za(i, _):
            for u in range(8):
                acc_v[pl.ds((i * 8 + u) * L, L)] = zv
            return 0
        lax.fori_loop(0, a2 // L // 8, za, 0)
        for u in range(a2 // L % 8):
            acc_v[pl.ds((a2 // L - a2 // L % 8 + u) * L, L)] = zv

        td.wait()
        lo = wid * cb

        def row(r, _):
            for v in range(CK // L):
                sv = src_sl[r, pl.ds(v * L, L)]
                dv = dst_sl[r, pl.ds(v * L, L)]
                vals = plsc.load_gather(t_v, [sv])
                plsc.addupdate_scatter(acc_v, [dv], vals)
            return 0

        def slab(k2, _):
            off = lo + k2 * sl
            pltpu.sync_copy(ei_hbm.at[0, pl.ds(off, sl)], src_sl)
            pltpu.sync_copy(ei_hbm.at[1, pl.ds(off, sl)], dst_sl)
            lax.fori_loop(0, sl, row, 0)
            return 0
        lax.fori_loop(0, cb // sl, slab, 0)

        if grem:
            @pl.when(wid < grem)
            def _extra():
                xo = NW * cb + wid * 8
                pltpu.sync_copy(ei_hbm.at[0, pl.ds(xo, 8)],
                                src_sl.at[pl.ds(0, 8)])
                pltpu.sync_copy(ei_hbm.at[1, pl.ds(xo, 8)],
                                dst_sl.at[pl.ds(0, 8)])
                lax.fori_loop(0, 8, row, 0)

        if tail:
            @pl.when(wid == grem)
            def _tail():
                pltpu.sync_copy(ei_hbm.at[0, pl.ds(tail0, tail)],
                                src_sl.at[pl.ds(0, tail)])
                pltpu.sync_copy(ei_hbm.at[1, pl.ds(tail0, tail)],
                                dst_sl.at[pl.ds(0, tail)])
                lax.fori_loop(0, tail, row, 0)

        pltpu.sync_copy(acc_v.at[pl.ds(0, n)], out_hbm.at[wid, 0])

    return k(t, ei_r)


def _dense_mid(psum_p, den_p, x_p, wl1k, bl1k, wr1k, wl2k, wr2k, bl2k):
    n8 = x_p.shape[0]

    def body(p_ref, den_ref, x_ref, wl1_ref, bl1_ref, wr1_ref, wl2_ref,
             wr2_ref, bl2_ref, t_ref, r2_ref):
        agg = (p_ref[0] + p_ref[1]) / den_ref[...]
        y = (jnp.dot(agg, wl1_ref[...], preferred_element_type=jnp.float32)
             + bl1_ref[...]
             + jnp.dot(x_ref[...], wr1_ref[...],
                       preferred_element_type=jnp.float32))
        y = jnp.maximum(y, 0.0)
        t_ref[...] = jnp.dot(y, wl2_ref[...],
                             preferred_element_type=jnp.float32)
        r2_ref[...] = (jnp.dot(y, wr2_ref[...],
                               preferred_element_type=jnp.float32)
                       + bl2_ref[...])

    return pl.pallas_call(
        body,
        out_shape=[
            jax.ShapeDtypeStruct((n8, 8), jnp.float32),
            jax.ShapeDtypeStruct((n8, 8), jnp.float32),
        ],
    )(psum_p, den_p, x_p, wl1k, bl1k, wr1k, wl2k, wr2k, bl2k)


def _dense_out(q2, pcnt2, r2row):
    n = q2.shape[1]

    def body(q_ref, c_ref, r2_ref, o_ref):
        q = jnp.sum(q_ref[...], axis=0, keepdims=True)
        den = jnp.maximum(jnp.sum(c_ref[...], axis=0), 1.0)
        o_ref[...] = q / den + r2_ref[...]

    return pl.pallas_call(
        body,
        out_shape=jax.ShapeDtypeStruct((1, n), jnp.float32),
    )(q2, pcnt2, r2row)


def kernel(edge_index, user_emb, movie_emb, W_l1, b_l1, W_r1, W_l2, b_l2,
           W_r2):
    x = jnp.concatenate([user_emb, movie_emb], axis=0)
    n = x.shape[0]
    e = edge_index.shape[1]

    ei = edge_index.astype(jnp.int32)
    if e % CK:
        pad = CK - e % CK
        ei = jnp.concatenate(
            [ei, jnp.stack([jnp.zeros((pad,), jnp.int32),
                            jnp.full((pad,), n, jnp.int32)])], axis=1)
    ei_r = ei.reshape(2, -1, CK)

    np_rows = (n + 1 + NS * L - 1) // (NS * L) * (NS * L)

    psum, pcnt = _seg_sum_rows(x, ei_r, np_rows)

    eye = jnp.eye(8, dtype=jnp.float32)
    denrow = jnp.maximum(jnp.sum(pcnt[:, 0, :], axis=0), 1.0)
    den_p = jnp.broadcast_to(denrow[:, None], (n, D)).reshape(n // 8, 8 * D)
    t8, r28 = _dense_mid(
        psum.reshape(NC, n // 8, 8 * D), den_p, x.reshape(n // 8, 8 * D),
        jnp.kron(eye, W_l1), jnp.tile(b_l1, 8).reshape(1, 8 * D),
        jnp.kron(eye, W_r1), jnp.kron(eye, W_l2), jnp.kron(eye, W_r2),
        jnp.broadcast_to(b_l2.reshape(1, 1), (1, 8)))

    q = _seg_sum_scalars(t8.reshape(n), ei_r).reshape(NW, n)
    out = _dense_out(q, pcnt, r28.reshape(1, n))
    return out.reshape(n, 1)
